# baseline (speedup 1.0000x reference)
import functools

import jax
import jax.numpy as jnp
from jax import lax
from jax.experimental import pallas as pl
from jax.experimental.pallas import tpu as pltpu, tpu_sc as plsc

VOCAB = 1_000_000
D = 64
N_VALUES = 4096
VAL_LEN = 50
Q_LEN = 200

NC, NS = 2, 16
NW = NC * NS
VALS_PER_TILE = N_VALUES // NW
ROWS_PER_TILE = VALS_PER_TILE * VAL_LEN
G = 100
IDX_ROWS = ROWS_PER_TILE // G
STEP_VALS = 16
STEP_ROWS = STEP_VALS * VAL_LEN
GATHERS_PER_STEP = STEP_ROWS // G
N_STEPS = VALS_PER_TILE // STEP_VALS


def _sc_pool(table, filt2, qt2, vt3):
    mesh = plsc.VectorSubcoreMesh(
        core_axis_name="c", subcore_axis_name="s",
        num_cores=NC, num_subcores=NS)

    @functools.partial(
        pl.kernel,
        out_type=[
            jax.ShapeDtypeStruct((N_VALUES, D), jnp.float32),
            jax.ShapeDtypeStruct((D,), jnp.float32),
        ],
        mesh=mesh,
        compiler_params=pltpu.CompilerParams(
            needs_layout_passes=False, use_tc_tiling_on_sc=False),
        scratch_types=[
            pltpu.VMEM((IDX_ROWS, G), jnp.int32),
            pltpu.VMEM((STEP_ROWS, D), jnp.float32),
            pltpu.VMEM((STEP_ROWS, D), jnp.float32),
            pltpu.VMEM((STEP_ROWS, 1), jnp.float32),
            pltpu.VMEM((STEP_ROWS, 1), jnp.float32),
            pltpu.VMEM((STEP_VALS, D), jnp.float32),
            pltpu.VMEM((2, G), jnp.int32),
            pltpu.SemaphoreType.DMA,
            pltpu.SemaphoreType.DMA,
        ],
    )
    def body(table_h, filt_h, qt_h, vt_h, vm_out, q_out,
             idx_v, rows0, rows1, w0, w1, stage, qidx, sem0, sem1):
        wid = lax.axis_index("s") * NC + lax.axis_index("c")
        zero = jnp.zeros((16,), jnp.float32)

        @pl.when(wid == 0)
        def _query():
            pltpu.sync_copy(qt_h, qidx)
            h1 = pltpu.async_copy(table_h.at[qidx.at[0]],
                                  rows0.at[pl.ds(0, G)], sem0)
            h2 = pltpu.async_copy(table_h.at[qidx.at[1]],
                                  rows0.at[pl.ds(G, G)], sem0)
            h1.wait()
            h2.wait()

            def qbody(l, accs):
                return tuple(accs[c] + rows0[l, pl.ds(c * 16, 16)]
                             for c in range(4))
            accs = lax.fori_loop(0, Q_LEN, qbody, (zero,) * 4)
            for c in range(4):
                stage[0, pl.ds(c * 16, 16)] = accs[c]
            pltpu.sync_copy(stage.at[0], q_out)

        pltpu.sync_copy(vt_h.at[wid], idx_v)
        bufs = [(rows0, w0, sem0), (rows1, w1, sem1)]

        def issue(s, rows_b, w_b, sem):
            handles = []
            for g in range(GATHERS_PER_STEP):
                r = s * GATHERS_PER_STEP + g
                handles.append(pltpu.async_copy(
                    table_h.at[idx_v.at[r]],
                    rows_b.at[pl.ds(g * G, G)], sem))
                handles.append(pltpu.async_copy(
                    filt_h.at[idx_v.at[r]],
                    w_b.at[pl.ds(g * G, G)], sem))
            return handles

        iota16 = lax.iota(jnp.int32, 16)
        vrow_base = iota16 * VAL_LEN
        col0 = jnp.zeros((16,), jnp.int32)

        def compute(s, rows_b, w_b):
            for dc in range(4):
                def lbody(l, accs):
                    row_idx = vrow_base + l
                    wv = plsc.load_gather(w_b, [row_idx, col0])
                    out = []
                    for dj in range(16):
                        col = jnp.full((16,), dc * 16 + dj, jnp.int32)
                        rv = plsc.load_gather(rows_b, [row_idx, col])
                        out.append(accs[dj] + wv * rv)
                    return tuple(out)
                accs = lax.fori_loop(0, VAL_LEN, lbody, (zero,) * 16)
                for dj in range(16):
                    col = jnp.full((16,), dc * 16 + dj, jnp.int32)
                    plsc.store_scatter(stage, [iota16, col],
                                       accs[dj] * (1.0 / VAL_LEN))
            pltpu.sync_copy(
                stage,
                vm_out.at[pl.ds(wid * VALS_PER_TILE + s * STEP_VALS,
                                STEP_VALS)])

        pending = issue(0, *bufs[0])
        for s in range(N_STEPS):
            nxt = issue(s + 1, *bufs[(s + 1) % 2]) if s + 1 < N_STEPS else []
            for h in pending:
                h.wait()
            rows_b, w_b, _sem = bufs[s % 2]
            compute(s, rows_b, w_b)
            pending = nxt

    return body(table, filt2, qt2, vt3)


def _tc_score_body(vm_ref, q_ref, o_ref):
    vm = vm_ref[...]
    q = q_ref[...]
    x = jnp.sum(vm * q, axis=1, keepdims=True)
    a = jnp.abs(x)
    m = jnp.max(a, axis=(0, 1), keepdims=True)
    e = jnp.exp(a - m)
    ssum = jnp.sum(e, axis=(0, 1), keepdims=True)
    o_ref[...] = (a - m) - jnp.log(ssum)


def kernel(table, filter_w, query_tokens, values_tokens):
    vt3 = values_tokens.reshape(NW, IDX_ROWS, G).astype(jnp.int32)
    filt2 = filter_w.reshape(VOCAB, 1)
    qt2 = query_tokens.reshape(2, G).astype(jnp.int32)

    pooled, q_enc = _sc_pool(table, filt2, qt2, vt3)

    out = pl.pallas_call(
        _tc_score_body,
        out_shape=jax.ShapeDtypeStruct((N_VALUES, 1), jnp.float32),
    )(pooled, q_enc.reshape(1, D))
    return out.reshape(N_VALUES)


# speedup vs baseline: 2.0216x; 2.0216x over previous
import functools

import jax
import jax.numpy as jnp
from jax import lax
from jax.experimental import pallas as pl
from jax.experimental.pallas import tpu as pltpu, tpu_sc as plsc

VOCAB = 1_000_000
D = 64
N_VALUES = 4096
VAL_LEN = 50
Q_LEN = 200

NC, NS = 2, 16
NW = NC * NS
VALS_PER_TILE = N_VALUES // NW
ROWS_PER_TILE = VALS_PER_TILE * VAL_LEN
G = 128
IDX_ROWS = ROWS_PER_TILE // G

MV_BLOCK = 4000
MV_GRID = VOCAB // MV_BLOCK


def _qenc_body(qtok_ref, table_ref, o_ref, buf, sem):
    def issue(i, c):
        t = qtok_ref[i]
        pltpu.make_async_copy(
            table_ref.at[pl.ds(t, 1), :], buf.at[pl.ds(i, 1), :], sem
        ).start()
        return c

    lax.fori_loop(0, Q_LEN, issue, 0)

    def drain(i, c):
        pltpu.make_async_copy(
            table_ref.at[pl.ds(0, 1), :], buf.at[pl.ds(0, 1), :], sem
        ).wait()
        return c

    lax.fori_loop(0, Q_LEN, drain, 0)
    o_ref[...] = jnp.sum(buf[...], axis=0, keepdims=True)


def _matvec_body(table_ref, filt_ref, q_ref, o_ref):
    s = lax.dot_general(
        q_ref[...], table_ref[...],
        (((1,), (1,)), ((), ())),
        precision=lax.Precision.HIGHEST,
        preferred_element_type=jnp.float32,
    )
    o_ref[...] = filt_ref[...] * s[None]


def _sc_pool(g_flat, vt3):
    mesh = plsc.VectorSubcoreMesh(
        core_axis_name="c", subcore_axis_name="s",
        num_cores=NC, num_subcores=NS)

    @functools.partial(
        pl.kernel,
        out_type=jax.ShapeDtypeStruct((N_VALUES,), jnp.float32),
        mesh=mesh,
        compiler_params=pltpu.CompilerParams(
            needs_layout_passes=False, use_tc_tiling_on_sc=False),
        scratch_types=[
            pltpu.VMEM((IDX_ROWS, G), jnp.int32),
            pltpu.VMEM((IDX_ROWS, G), jnp.float32),
            pltpu.VMEM((VALS_PER_TILE,), jnp.float32),
            pltpu.SemaphoreType.DMA,
        ],
    )
    def body(g_h, vt_h, x_out, idx_v, w_v, xout, sem):
        wid = lax.axis_index("s") * NC + lax.axis_index("c")
        pltpu.sync_copy(vt_h.at[wid], idx_v)
        handles = [
            pltpu.async_copy(g_h.at[idx_v.at[r]], w_v.at[r], sem)
            for r in range(IDX_ROWS)
        ]
        for h in handles:
            h.wait()

        iota16 = lax.iota(jnp.int32, 16)
        zero = jnp.zeros((16,), jnp.float32)
        for gi in range(VALS_PER_TILE // 16):
            base = gi * 16 * VAL_LEN + iota16 * VAL_LEN

            def lbody(l, acc):
                fl = base + l
                wv = plsc.load_gather(w_v, [fl >> 7, fl & 127])
                return acc + wv

            acc = lax.fori_loop(0, VAL_LEN, lbody, zero)
            xout[pl.ds(gi * 16, 16)] = acc * (1.0 / VAL_LEN)
        pltpu.sync_copy(xout, x_out.at[pl.ds(wid * VALS_PER_TILE,
                                             VALS_PER_TILE)])

    return body(g_flat, vt3)


def _softmax_body(x_ref, o_ref):
    a = jnp.abs(x_ref[...])
    m = jnp.max(a, axis=(0, 1), keepdims=True)
    e = jnp.exp(a - m)
    ssum = jnp.sum(e, axis=(0, 1), keepdims=True)
    o_ref[...] = (a - m) - jnp.log(ssum)


def kernel(table, filter_w, query_tokens, values_tokens):
    vt3 = values_tokens.reshape(NW, IDX_ROWS, G).astype(jnp.int32)
    filt2 = filter_w.reshape(MV_GRID, 1, MV_BLOCK)
    qtok = query_tokens.astype(jnp.int32)

    q_enc = pl.pallas_call(
        _qenc_body,
        in_specs=[
            pl.BlockSpec(memory_space=pltpu.SMEM),
            pl.BlockSpec(memory_space=pltpu.HBM),
        ],
        out_shape=jax.ShapeDtypeStruct((1, D), jnp.float32),
        scratch_shapes=[
            pltpu.VMEM((Q_LEN, D), jnp.float32),
            pltpu.SemaphoreType.DMA,
        ],
    )(qtok, table)

    g2 = pl.pallas_call(
        _matvec_body,
        grid=(MV_GRID,),
        in_specs=[
            pl.BlockSpec((MV_BLOCK, D), lambda i: (i, 0)),
            pl.BlockSpec((1, 1, MV_BLOCK), lambda i: (i, 0, 0)),
            pl.BlockSpec((1, D), lambda i: (0, 0)),
        ],
        out_specs=pl.BlockSpec((1, 1, MV_BLOCK), lambda i: (i, 0, 0)),
        out_shape=jax.ShapeDtypeStruct((MV_GRID, 1, MV_BLOCK), jnp.float32),
    )(table, filt2, q_enc)

    g_flat = g2.reshape(VOCAB)
    x = _sc_pool(g_flat, vt3)

    out = pl.pallas_call(
        _softmax_body,
        out_shape=jax.ShapeDtypeStruct((32, 128), jnp.float32),
    )(x.reshape(32, 128))
    return out.reshape(N_VALUES)


# speedup vs baseline: 2.8000x; 1.3851x over previous
import functools

import jax
import jax.numpy as jnp
from jax import lax
from jax.experimental import pallas as pl
from jax.experimental.pallas import tpu as pltpu, tpu_sc as plsc

VOCAB = 1_000_000
D = 64
N_VALUES = 4096
VAL_LEN = 50
Q_LEN = 200

NC, NS = 2, 16
NW = NC * NS
VALS_PER_TILE = N_VALUES // NW
ROWS_PER_TILE = VALS_PER_TILE * VAL_LEN
G = 128
IDX_ROWS = ROWS_PER_TILE // G

MV_BLOCK = 8000
MV_GRID = VOCAB // MV_BLOCK


def _qenc_body(qtok_ref, table_ref, o_ref, buf, sem):
    def issue(i, c):
        t = qtok_ref[i]
        pltpu.make_async_copy(
            table_ref.at[pl.ds(t, 1), :], buf.at[pl.ds(i, 1), :], sem
        ).start()
        return c

    lax.fori_loop(0, Q_LEN, issue, 0)

    def drain(i, c):
        pltpu.make_async_copy(
            table_ref.at[pl.ds(0, 1), :], buf.at[pl.ds(0, 1), :], sem
        ).wait()
        return c

    lax.fori_loop(0, Q_LEN, drain, 0)
    o_ref[...] = jnp.sum(buf[...], axis=0, keepdims=True)


def _matvec_body(table_ref, filt_ref, q_ref, o_ref):
    s = lax.dot_general(
        q_ref[...], table_ref[...],
        (((1,), (1,)), ((), ())),
        preferred_element_type=jnp.float32,
    )
    o_ref[...] = filt_ref[...] * s[None]


def _sc_pool(g_flat, vt3):
    mesh = plsc.VectorSubcoreMesh(
        core_axis_name="c", subcore_axis_name="s",
        num_cores=NC, num_subcores=NS)

    @functools.partial(
        pl.kernel,
        out_type=jax.ShapeDtypeStruct((N_VALUES,), jnp.float32),
        mesh=mesh,
        compiler_params=pltpu.CompilerParams(
            needs_layout_passes=False, use_tc_tiling_on_sc=False),
        scratch_types=[
            pltpu.VMEM((IDX_ROWS, G), jnp.int32),
            pltpu.VMEM((IDX_ROWS, G), jnp.float32),
            pltpu.VMEM((VALS_PER_TILE,), jnp.float32),
            pltpu.SemaphoreType.DMA,
        ],
    )
    def body(g_h, vt_h, x_out, idx_v, w_v, xout, sem):
        wid = lax.axis_index("s") * NC + lax.axis_index("c")
        pltpu.sync_copy(vt_h.at[wid], idx_v)
        handles = [
            pltpu.async_copy(g_h.at[idx_v.at[r]], w_v.at[r], sem)
            for r in range(IDX_ROWS)
        ]
        for h in handles:
            h.wait()

        iota16 = lax.iota(jnp.int32, 16)
        zero = jnp.zeros((16,), jnp.float32)
        for gi in range(VALS_PER_TILE // 16):
            base = gi * 16 * VAL_LEN + iota16 * VAL_LEN

            def lbody(l, acc):
                fl = base + l
                wv = plsc.load_gather(w_v, [fl >> 7, fl & 127])
                return acc + wv

            acc = lax.fori_loop(0, VAL_LEN, lbody, zero)
            xout[pl.ds(gi * 16, 16)] = acc * (1.0 / VAL_LEN)
        pltpu.sync_copy(xout, x_out.at[pl.ds(wid * VALS_PER_TILE,
                                             VALS_PER_TILE)])

    return body(g_flat, vt3)


def _softmax_body(x_ref, o_ref):
    a = jnp.abs(x_ref[...])
    m = jnp.max(a, axis=(0, 1), keepdims=True)
    e = jnp.exp(a - m)
    ssum = jnp.sum(e, axis=(0, 1), keepdims=True)
    o_ref[...] = (a - m) - jnp.log(ssum)


def kernel(table, filter_w, query_tokens, values_tokens):
    vt3 = values_tokens.reshape(NW, IDX_ROWS, G).astype(jnp.int32)
    filt2 = filter_w.reshape(MV_GRID, 1, MV_BLOCK)
    qtok = query_tokens.astype(jnp.int32)

    q_enc = pl.pallas_call(
        _qenc_body,
        in_specs=[
            pl.BlockSpec(memory_space=pltpu.SMEM),
            pl.BlockSpec(memory_space=pltpu.HBM),
        ],
        out_shape=jax.ShapeDtypeStruct((1, D), jnp.float32),
        scratch_shapes=[
            pltpu.VMEM((Q_LEN, D), jnp.float32),
            pltpu.SemaphoreType.DMA,
        ],
    )(qtok, table)

    g2 = pl.pallas_call(
        _matvec_body,
        grid=(MV_GRID,),
        in_specs=[
            pl.BlockSpec((MV_BLOCK, D), lambda i: (i, 0)),
            pl.BlockSpec((1, 1, MV_BLOCK), lambda i: (i, 0, 0)),
            pl.BlockSpec((1, D), lambda i: (0, 0)),
        ],
        out_specs=pl.BlockSpec((1, 1, MV_BLOCK), lambda i: (i, 0, 0)),
        out_shape=jax.ShapeDtypeStruct((MV_GRID, 1, MV_BLOCK), jnp.float32),
    )(table, filt2, q_enc)

    g_flat = g2.reshape(VOCAB)
    x = _sc_pool(g_flat, vt3)

    out = pl.pallas_call(
        _softmax_body,
        out_shape=jax.ShapeDtypeStruct((32, 128), jnp.float32),
    )(x.reshape(32, 128))
    return out.reshape(N_VALUES)


# speedup vs baseline: 2.9777x; 1.0635x over previous
import functools

import jax
import jax.numpy as jnp
from jax import lax
from jax.experimental import pallas as pl
from jax.experimental.pallas import tpu as pltpu, tpu_sc as plsc

VOCAB = 1_000_000
D = 64
N_VALUES = 4096
VAL_LEN = 50
Q_LEN = 200

NC, NS = 2, 16
NW = NC * NS
VALS_PER_TILE = N_VALUES // NW
ROWS_PER_TILE = VALS_PER_TILE * VAL_LEN
G = 128
IDX_ROWS = ROWS_PER_TILE // G

MV_BLOCK = 40000
MV_GRID = VOCAB // MV_BLOCK


def _qenc_body(qtok_ref, table_ref, o_ref, buf, sem):
    def issue(i, c):
        t = qtok_ref[i]
        pltpu.make_async_copy(
            table_ref.at[pl.ds(t, 1), :], buf.at[pl.ds(i, 1), :], sem
        ).start()
        return c

    lax.fori_loop(0, Q_LEN, issue, 0)

    def drain(i, c):
        pltpu.make_async_copy(
            table_ref.at[pl.ds(0, 1), :], buf.at[pl.ds(0, 1), :], sem
        ).wait()
        return c

    lax.fori_loop(0, Q_LEN, drain, 0)
    o_ref[...] = jnp.sum(buf[...], axis=0, keepdims=True)


def _matvec_body(table_ref, filt_ref, q_ref, o_ref):
    s = lax.dot_general(
        q_ref[...], table_ref[...],
        (((1,), (1,)), ((), ())),
        preferred_element_type=jnp.float32,
    )
    o_ref[...] = filt_ref[...] * s[None]


def _sc_pool(g_flat, vt3):
    mesh = plsc.VectorSubcoreMesh(
        core_axis_name="c", subcore_axis_name="s",
        num_cores=NC, num_subcores=NS)

    @functools.partial(
        pl.kernel,
        out_type=jax.ShapeDtypeStruct((N_VALUES,), jnp.float32),
        mesh=mesh,
        compiler_params=pltpu.CompilerParams(
            needs_layout_passes=False, use_tc_tiling_on_sc=False),
        scratch_types=[
            pltpu.VMEM((IDX_ROWS, G), jnp.int32),
            pltpu.VMEM((IDX_ROWS, G), jnp.float32),
            pltpu.VMEM((VALS_PER_TILE,), jnp.float32),
            pltpu.SemaphoreType.DMA,
        ],
    )
    def body(g_h, vt_h, x_out, idx_v, w_v, xout, sem):
        wid = lax.axis_index("s") * NC + lax.axis_index("c")
        pltpu.sync_copy(vt_h.at[wid], idx_v)
        handles = [
            pltpu.async_copy(g_h.at[idx_v.at[r]], w_v.at[r], sem)
            for r in range(IDX_ROWS)
        ]
        for h in handles:
            h.wait()

        iota16 = lax.iota(jnp.int32, 16)
        zero = jnp.zeros((16,), jnp.float32)
        for gi in range(VALS_PER_TILE // 16):
            base = gi * 16 * VAL_LEN + iota16 * VAL_LEN

            def lbody(l, acc):
                fl = base + l
                wv = plsc.load_gather(w_v, [fl >> 7, fl & 127])
                return acc + wv

            acc = lax.fori_loop(0, VAL_LEN, lbody, zero)
            xout[pl.ds(gi * 16, 16)] = acc * (1.0 / VAL_LEN)
        pltpu.sync_copy(xout, x_out.at[pl.ds(wid * VALS_PER_TILE,
                                             VALS_PER_TILE)])

    return body(g_flat, vt3)


def _softmax_body(x_ref, o_ref):
    a = jnp.abs(x_ref[...])
    m = jnp.max(a, axis=(0, 1), keepdims=True)
    e = jnp.exp(a - m)
    ssum = jnp.sum(e, axis=(0, 1), keepdims=True)
    o_ref[...] = (a - m) - jnp.log(ssum)


def kernel(table, filter_w, query_tokens, values_tokens):
    vt3 = values_tokens.reshape(NW, IDX_ROWS, G).astype(jnp.int32)
    filt2 = filter_w.reshape(MV_GRID, 1, MV_BLOCK)
    qtok = query_tokens.astype(jnp.int32)

    q_enc = pl.pallas_call(
        _qenc_body,
        in_specs=[
            pl.BlockSpec(memory_space=pltpu.SMEM),
            pl.BlockSpec(memory_space=pltpu.HBM),
        ],
        out_shape=jax.ShapeDtypeStruct((1, D), jnp.float32),
        scratch_shapes=[
            pltpu.VMEM((Q_LEN, D), jnp.float32),
            pltpu.SemaphoreType.DMA,
        ],
    )(qtok, table)

    g2 = pl.pallas_call(
        _matvec_body,
        grid=(MV_GRID,),
        in_specs=[
            pl.BlockSpec((MV_BLOCK, D), lambda i: (i, 0)),
            pl.BlockSpec((1, 1, MV_BLOCK), lambda i: (i, 0, 0)),
            pl.BlockSpec((1, D), lambda i: (0, 0)),
        ],
        out_specs=pl.BlockSpec((1, 1, MV_BLOCK), lambda i: (i, 0, 0)),
        out_shape=jax.ShapeDtypeStruct((MV_GRID, 1, MV_BLOCK), jnp.float32),
    )(table, filt2, q_enc)

    g_flat = g2.reshape(VOCAB)
    x = _sc_pool(g_flat, vt3)

    out = pl.pallas_call(
        _softmax_body,
        out_shape=jax.ShapeDtypeStruct((32, 128), jnp.float32),
    )(x.reshape(32, 128))
    return out.reshape(N_VALUES)


# speedup vs baseline: 3.1587x; 1.0608x over previous
import functools

import jax
import jax.numpy as jnp
from jax import lax
from jax.experimental import pallas as pl
from jax.experimental.pallas import tpu as pltpu, tpu_sc as plsc

VOCAB = 1_000_000
D = 64
N_VALUES = 4096
VAL_LEN = 50
Q_LEN = 200

NC, NS = 2, 16
NW = NC * NS
VALS_PER_TILE = N_VALUES // NW
ROWS_PER_TILE = VALS_PER_TILE * VAL_LEN
G = 128
IDX_ROWS = ROWS_PER_TILE // G

MV_CHUNK = 12500
MV_NCHUNK = VOCAB // MV_CHUNK
MV_NBUF = 6


def _qenc_body(qtok_ref, table_ref, o_ref, buf, sem):
    def issue(i, c):
        t = qtok_ref[i]
        pltpu.make_async_copy(
            table_ref.at[pl.ds(t, 1), :], buf.at[pl.ds(i, 1), :], sem
        ).start()
        return c

    lax.fori_loop(0, Q_LEN, issue, 0)

    def drain(i, c):
        pltpu.make_async_copy(
            table_ref.at[pl.ds(0, 1), :], buf.at[pl.ds(0, 1), :], sem
        ).wait()
        return c

    lax.fori_loop(0, Q_LEN, drain, 0)
    o_ref[...] = jnp.sum(buf[...], axis=0, keepdims=True)


def _matvec_body(table_hbm, filt_ref, q_ref, o_ref, buf, sems):
    def start(c, slot):
        pltpu.make_async_copy(
            table_hbm.at[pl.ds(c * MV_CHUNK, MV_CHUNK), :],
            buf.at[slot], sems.at[slot],
        ).start()

    for i in range(MV_NBUF):
        start(i, i)

    def step(c, carry):
        slot = lax.rem(c, MV_NBUF)
        pltpu.make_async_copy(
            table_hbm.at[pl.ds(0, MV_CHUNK), :], buf.at[slot], sems.at[slot]
        ).wait()
        s = lax.dot_general(
            q_ref[...], buf[slot],
            (((1,), (1,)), ((), ())),
            preferred_element_type=jnp.float32,
        )
        o_ref[pl.ds(c, 1), :] = filt_ref[pl.ds(c, 1), :] * s

        @pl.when(c + MV_NBUF < MV_NCHUNK)
        def _():
            start(c + MV_NBUF, slot)

        return carry

    lax.fori_loop(0, MV_NCHUNK, step, 0)


def _sc_pool(g_flat, vt3):
    mesh = plsc.VectorSubcoreMesh(
        core_axis_name="c", subcore_axis_name="s",
        num_cores=NC, num_subcores=NS)

    @functools.partial(
        pl.kernel,
        out_type=jax.ShapeDtypeStruct((N_VALUES,), jnp.float32),
        mesh=mesh,
        compiler_params=pltpu.CompilerParams(
            needs_layout_passes=False, use_tc_tiling_on_sc=False),
        scratch_types=[
            pltpu.VMEM((IDX_ROWS, G), jnp.int32),
            pltpu.VMEM((IDX_ROWS, G), jnp.float32),
            pltpu.VMEM((VALS_PER_TILE,), jnp.float32),
            pltpu.SemaphoreType.DMA,
        ],
    )
    def body(g_h, vt_h, x_out, idx_v, w_v, xout, sem):
        wid = lax.axis_index("s") * NC + lax.axis_index("c")
        pltpu.sync_copy(vt_h.at[wid], idx_v)
        handles = [
            pltpu.async_copy(g_h.at[idx_v.at[r]], w_v.at[r], sem)
            for r in range(IDX_ROWS)
        ]
        for h in handles:
            h.wait()

        iota16 = lax.iota(jnp.int32, 16)
        zero = jnp.zeros((16,), jnp.float32)
        for gi in range(VALS_PER_TILE // 16):
            base = gi * 16 * VAL_LEN + iota16 * VAL_LEN

            def lbody(l, acc):
                fl = base + l
                wv = plsc.load_gather(w_v, [fl >> 7, fl & 127])
                return acc + wv

            acc = lax.fori_loop(0, VAL_LEN, lbody, zero)
            xout[pl.ds(gi * 16, 16)] = acc * (1.0 / VAL_LEN)
        pltpu.sync_copy(xout, x_out.at[pl.ds(wid * VALS_PER_TILE,
                                             VALS_PER_TILE)])

    return body(g_flat, vt3)


def _softmax_body(x_ref, o_ref):
    a = jnp.abs(x_ref[...])
    m = jnp.max(a, axis=(0, 1), keepdims=True)
    e = jnp.exp(a - m)
    ssum = jnp.sum(e, axis=(0, 1), keepdims=True)
    o_ref[...] = (a - m) - jnp.log(ssum)


def kernel(table, filter_w, query_tokens, values_tokens):
    vt3 = values_tokens.reshape(NW, IDX_ROWS, G).astype(jnp.int32)
    filt2 = filter_w.reshape(MV_NCHUNK, MV_CHUNK)
    qtok = query_tokens.astype(jnp.int32)

    q_enc = pl.pallas_call(
        _qenc_body,
        in_specs=[
            pl.BlockSpec(memory_space=pltpu.SMEM),
            pl.BlockSpec(memory_space=pltpu.HBM),
        ],
        out_shape=jax.ShapeDtypeStruct((1, D), jnp.float32),
        scratch_shapes=[
            pltpu.VMEM((Q_LEN, D), jnp.float32),
            pltpu.SemaphoreType.DMA,
        ],
    )(qtok, table)

    g2 = pl.pallas_call(
        _matvec_body,
        in_specs=[
            pl.BlockSpec(memory_space=pltpu.HBM),
            pl.BlockSpec(memory_space=pltpu.VMEM),
            pl.BlockSpec(memory_space=pltpu.VMEM),
        ],
        out_shape=jax.ShapeDtypeStruct((MV_NCHUNK, MV_CHUNK), jnp.float32),
        scratch_shapes=[
            pltpu.VMEM((MV_NBUF, MV_CHUNK, D), jnp.float32),
            pltpu.SemaphoreType.DMA((MV_NBUF,)),
        ],
    )(table, filt2, q_enc)

    g_flat = g2.reshape(VOCAB)
    x = _sc_pool(g_flat, vt3)

    out = pl.pallas_call(
        _softmax_body,
        out_shape=jax.ShapeDtypeStruct((32, 128), jnp.float32),
    )(x.reshape(32, 128))
    return out.reshape(N_VALUES)
